# baseline (speedup 1.0000x reference)
import functools

import jax
import jax.numpy as jnp
from jax import lax
from jax.experimental import pallas as pl
from jax.experimental.pallas import tpu as pltpu
from jax.experimental.pallas import tpu_sc as plsc

N = 10000
E = 160000
D = 256
H = 16
C = 16
HC = 256
TD = 16
HALF = 128

NC = 2
NS = 16
NW = NC * NS
B = 128
NCHUNK = E // B
KMAX = -(-NCHUNK // NW)
B2 = 64
NCHUNK2 = E // B2
KMAX2 = 80
NP = NW * 314
ZROWS = NP // NS

_mesh = plsc.VectorSubcoreMesh(core_axis_name="c", subcore_axis_name="s",
                               num_cores=NC, num_subcores=NS)


def _node_body(x_ref, wlt_ref, wrt_ref, plm_ref, prm_ref,
               xw0_ref, xw1_ref, res_ref, al_ref, ar_ref):
    xb = x_ref[...]
    xw = jnp.dot(xb, wlt_ref[...], preferred_element_type=jnp.float32)
    res_ref[...] = jnp.dot(xb, wrt_ref[...], preferred_element_type=jnp.float32)
    xw0_ref[...] = xw[:, :HALF]
    xw1_ref[...] = xw[:, HALF:]
    al_ref[...] = jnp.dot(xw, plm_ref[...], preferred_element_type=jnp.float32)
    ar_ref[...] = jnp.dot(xw, prm_ref[...], preferred_element_type=jnp.float32)


def _ab_body(ea_ref, ew_ref, m_ref, c0_ref, aev_ref, ab_ref):
    ab_ref[...] = (jnp.dot(ea_ref[...], m_ref[...],
                           preferred_element_type=jnp.float32)
                   + c0_ref[...] + ew_ref[...] * aev_ref[...])


def _ef_body(ea_ref, wet_ref, bev_ref, ef0_ref, ef1_ref):
    ef = jnp.dot(ea_ref[...], wet_ref[...], preferred_element_type=jnp.float32)
    ef = ef + bev_ref[...]
    ef0_ref[...] = ef[:, :HALF]
    ef1_ref[...] = ef[:, HALF:]


def _combine_body(a00_ref, a01_ref, a10_ref, a11_ref, ss_ref, rep_ref,
                  res_ref, out_ref):
    s0 = a00_ref[0] + a01_ref[0]
    s1 = a10_ref[0] + a11_ref[0]
    agg = jnp.concatenate([s0, s1], axis=1)
    den = jnp.dot(ss_ref[...], rep_ref[...],
                  preferred_element_type=jnp.float32) + 1e-16
    agg = agg / den
    out_ref[...] = jnp.where(agg > 0.0, agg, jnp.exp(agg) - 1.0) + res_ref[...]


def _sc_softmax_body(src_hbm, dst_hbm, al_hbm, ar_hbm, ab_hbm,
                     p_hbm, part_hbm,
                     isrc0, isrc1, idst0, idst1, gl0, gl1, gr0, gr1,
                     ab0, ab1, p_v, acc_sh, sem0, sem1):
    isrc_v = [isrc0, isrc1]
    idst_v = [idst0, idst1]
    gl_v = [gl0, gl1]
    gr_v = [gr0, gr1]
    ab_v = [ab0, ab1]
    sem = [sem0, sem1]
    cid = lax.axis_index("c")
    sid = lax.axis_index("s")
    wid = sid * NC + cid
    zf = jnp.zeros((H,), jnp.float32)

    def zinit(j, _):
        gl_v[0][j] = zf
        return ()
    lax.fori_loop(0, B, zinit, ())
    for q in range(4):
        pltpu.sync_copy(gl_v[0], acc_sh.at[pl.ds(sid * ZROWS + q * B, B)])
    pltpu.sync_copy(gl_v[0].at[pl.ds(0, ZROWS - 4 * B)],
                    acc_sh.at[pl.ds(sid * ZROWS + 4 * B, ZROWS - 4 * B)])
    plsc.subcore_barrier()

    def issue(k, b):
        g = k * NW + wid

        @pl.when(g < NCHUNK)
        def _():
            base = g * B
            pltpu.sync_copy(src_hbm.at[pl.ds(base, B)], isrc_v[b])
            pltpu.sync_copy(dst_hbm.at[pl.ds(base, B)], idst_v[b])
            pltpu.async_copy(al_hbm.at[isrc_v[b]], gl_v[b], sem[b])
            pltpu.async_copy(ar_hbm.at[idst_v[b]], gr_v[b], sem[b])
            pltpu.async_copy(ab_hbm.at[pl.ds(base, B)], ab_v[b], sem[b])

    def consume(k, b):
        g = k * NW + wid

        @pl.when(g < NCHUNK)
        def _():
            base = g * B
            pltpu.make_async_copy(al_hbm.at[isrc_v[b]], gl_v[b], sem[b]).wait()
            pltpu.make_async_copy(ar_hbm.at[idst_v[b]], gr_v[b], sem[b]).wait()
            pltpu.make_async_copy(ab_hbm.at[pl.ds(base, B)], ab_v[b],
                                  sem[b]).wait()

            def edge(j, _):
                s = gl_v[b][j] + gr_v[b][j] + ab_v[b][j]
                a = jnp.where(s >= 0.0, s, 0.2 * s)
                p_v[j] = jnp.exp(a)
                return ()
            lax.fori_loop(0, B, edge, ())

            pltpu.sync_copy(p_v, p_hbm.at[pl.ds(base, B)])
            pltpu.sync_copy(p_v, acc_sh.at[idst_v[b]], add=True)

    issue(0, 0)
    issue(1, 1)

    def pair(ii, _):
        for par in range(2):
            k = ii * 2 + par
            consume(k, par)
            issue(k + 2, par)
        return ()
    lax.fori_loop(0, KMAX // 2, pair, ())
    plsc.subcore_barrier()

    @pl.when(sid == 0)
    def _():
        pltpu.sync_copy(acc_sh, part_hbm.at[cid])


def _sc_agg_body(src_hbm, dst_hbm, p_hbm, xw0_hbm, xw1_hbm,
                 ef0_hbm, ef1_hbm, agg0_hbm, agg1_hbm,
                 isrc0, isrc1, idst0, idst1, xg0, xg1, ef0_v, ef1_v,
                 p0, p1, acc_sh, sem0, sem1):
    isrc_v = [isrc0, isrc1]
    idst_v = [idst0, idst1]
    xg_v = [xg0, xg1]
    ef_v = [ef0_v, ef1_v]
    p_v = [p0, p1]
    sem = [sem0, sem1]
    cid = lax.axis_index("c")
    sid = lax.axis_index("s")
    wid = sid * NC + cid
    zf = jnp.zeros((C,), jnp.float32)

    for half in range(2):
        xwh = xw0_hbm if half == 0 else xw1_hbm
        efh = ef0_hbm if half == 0 else ef1_hbm
        aggh = agg0_hbm if half == 0 else agg1_hbm

        def zinit(j, _):
            for c8 in range(8):
                xg0[j, pl.ds(c8 * 16, 16)] = zf
            return ()
        lax.fori_loop(0, B2, zinit, ())
        for q in range(9):
            pltpu.sync_copy(xg0, acc_sh.at[pl.ds(sid * ZROWS + q * B2, B2)])
        pltpu.sync_copy(xg0.at[pl.ds(0, ZROWS - 9 * B2)],
                        acc_sh.at[pl.ds(sid * ZROWS + 9 * B2, ZROWS - 9 * B2)])
        plsc.subcore_barrier()

        def issue(k, b):
            g = k * NW + wid

            @pl.when(g < NCHUNK2)
            def _():
                base = g * B2
                pltpu.sync_copy(src_hbm.at[pl.ds(base, B2)], isrc_v[b])
                pltpu.sync_copy(dst_hbm.at[pl.ds(base, B2)], idst_v[b])
                pltpu.async_copy(xwh.at[isrc_v[b]], xg_v[b], sem[b])
                pltpu.async_copy(efh.at[pl.ds(base, B2)], ef_v[b], sem[b])
                pltpu.async_copy(p_hbm.at[pl.ds(base, B2)], p_v[b], sem[b])

        def consume(k, b):
            g = k * NW + wid

            @pl.when(g < NCHUNK2)
            def _():
                base = g * B2
                pltpu.make_async_copy(xwh.at[isrc_v[b]], xg_v[b],
                                      sem[b]).wait()
                pltpu.make_async_copy(efh.at[pl.ds(base, B2)], ef_v[b],
                                      sem[b]).wait()
                pltpu.make_async_copy(p_hbm.at[pl.ds(base, B2)], p_v[b],
                                      sem[b]).wait()

                def edge(j, _):
                    co = p_v[b][j]
                    for hh in range(8):
                        sc = co[half * 8 + hh]
                        sl = pl.ds(hh * 16, 16)
                        xg_v[b][j, sl] = (xg_v[b][j, sl] + ef_v[b][j, sl]) * sc
                    return ()
                lax.fori_loop(0, B2, edge, ())

                pltpu.sync_copy(xg_v[b], acc_sh.at[idst_v[b]], add=True)

        issue(0, 0)
        issue(1, 1)

        def pair(ii, _):
            for par in range(2):
                k = ii * 2 + par
                consume(k, par)
                issue(k + 2, par)
            return ()
        lax.fori_loop(0, KMAX2 // 2, pair, ())
        plsc.subcore_barrier()

        @pl.when(sid == 0)
        def _():
            pltpu.sync_copy(acc_sh, aggh.at[cid])
        plsc.subcore_barrier()


_sc_softmax = pl.kernel(
    _sc_softmax_body,
    out_type=(jax.ShapeDtypeStruct((E, H), jnp.float32),
              jax.ShapeDtypeStruct((NC, NP, H), jnp.float32)),
    mesh=_mesh,
    scratch_types=[
        pltpu.VMEM((B,), jnp.int32),
        pltpu.VMEM((B,), jnp.int32),
        pltpu.VMEM((B,), jnp.int32),
        pltpu.VMEM((B,), jnp.int32),
        pltpu.VMEM((B, H), jnp.float32),
        pltpu.VMEM((B, H), jnp.float32),
        pltpu.VMEM((B, H), jnp.float32),
        pltpu.VMEM((B, H), jnp.float32),
        pltpu.VMEM((B, H), jnp.float32),
        pltpu.VMEM((B, H), jnp.float32),
        pltpu.VMEM((B, H), jnp.float32),
        pltpu.VMEM_SHARED((NP, H), jnp.float32),
        pltpu.SemaphoreType.DMA,
        pltpu.SemaphoreType.DMA,
    ],
    compiler_params=pltpu.CompilerParams(use_tc_tiling_on_sc=False),
)

_sc_agg = pl.kernel(
    _sc_agg_body,
    out_type=(jax.ShapeDtypeStruct((NC, NP, HALF), jnp.float32),
              jax.ShapeDtypeStruct((NC, NP, HALF), jnp.float32)),
    mesh=_mesh,
    scratch_types=[
        pltpu.VMEM((B2,), jnp.int32),
        pltpu.VMEM((B2,), jnp.int32),
        pltpu.VMEM((B2,), jnp.int32),
        pltpu.VMEM((B2,), jnp.int32),
        pltpu.VMEM((B2, HALF), jnp.float32),
        pltpu.VMEM((B2, HALF), jnp.float32),
        pltpu.VMEM((B2, HALF), jnp.float32),
        pltpu.VMEM((B2, HALF), jnp.float32),
        pltpu.VMEM((B2, H), jnp.float32),
        pltpu.VMEM((B2, H), jnp.float32),
        pltpu.VMEM_SHARED((NP, HALF), jnp.float32),
        pltpu.SemaphoreType.DMA,
        pltpu.SemaphoreType.DMA,
    ],
)


@jax.jit
def kernel(x, edge_index, edge_weight, edge_attr, W_lin, att_l, att_r, att_e,
           W_e, b_e, W_res):
    src = edge_index[0]
    dst = edge_index[1]

    eye = jnp.eye(H, dtype=jnp.float32)
    plm = (att_l[0][:, :, None] * eye[:, None, :]).reshape(HC, H)
    prm = (att_r[0][:, :, None] * eye[:, None, :]).reshape(HC, H)
    aev = att_e.reshape(1, H)
    bev = b_e.reshape(1, HC)
    ew2 = edge_weight.reshape(E, 1)

    bn = 400
    xw0, xw1, res, al, ar = pl.pallas_call(
        _node_body,
        grid=(N // bn,),
        in_specs=[
            pl.BlockSpec((bn, D), lambda i: (i, 0)),
            pl.BlockSpec((D, HC), lambda i: (0, 0)),
            pl.BlockSpec((D, HC), lambda i: (0, 0)),
            pl.BlockSpec((HC, H), lambda i: (0, 0)),
            pl.BlockSpec((HC, H), lambda i: (0, 0)),
        ],
        out_specs=[
            pl.BlockSpec((bn, HALF), lambda i: (i, 0)),
            pl.BlockSpec((bn, HALF), lambda i: (i, 0)),
            pl.BlockSpec((bn, HC), lambda i: (i, 0)),
            pl.BlockSpec((bn, H), lambda i: (i, 0)),
            pl.BlockSpec((bn, H), lambda i: (i, 0)),
        ],
        out_shape=[
            jax.ShapeDtypeStruct((N, HALF), jnp.float32),
            jax.ShapeDtypeStruct((N, HALF), jnp.float32),
            jax.ShapeDtypeStruct((N, HC), jnp.float32),
            jax.ShapeDtypeStruct((N, H), jnp.float32),
            jax.ShapeDtypeStruct((N, H), jnp.float32),
        ],
    )(x, W_lin.T, W_res.T, plm, prm)

    wet = W_e.T
    m_comb = jnp.dot(wet, plm)
    c0 = jnp.dot(bev, plm)
    be = 2000
    ab = pl.pallas_call(
        _ab_body,
        grid=(E // be,),
        in_specs=[
            pl.BlockSpec((be, TD), lambda i: (i, 0)),
            pl.BlockSpec((be, 1), lambda i: (i, 0)),
            pl.BlockSpec((TD, H), lambda i: (0, 0)),
            pl.BlockSpec((1, H), lambda i: (0, 0)),
            pl.BlockSpec((1, H), lambda i: (0, 0)),
        ],
        out_specs=pl.BlockSpec((be, H), lambda i: (i, 0)),
        out_shape=jax.ShapeDtypeStruct((E, H), jnp.float32),
    )(edge_attr, ew2, m_comb, c0, aev)

    p, parts = _sc_softmax(src, dst, al, ar, ab)
    ss = parts[0] + parts[1]

    ef0, ef1 = pl.pallas_call(
        _ef_body,
        grid=(E // be,),
        in_specs=[
            pl.BlockSpec((be, TD), lambda i: (i, 0)),
            pl.BlockSpec((TD, HC), lambda i: (0, 0)),
            pl.BlockSpec((1, HC), lambda i: (0, 0)),
        ],
        out_specs=[
            pl.BlockSpec((be, HALF), lambda i: (i, 0)),
            pl.BlockSpec((be, HALF), lambda i: (i, 0)),
        ],
        out_shape=[
            jax.ShapeDtypeStruct((E, HALF), jnp.float32),
            jax.ShapeDtypeStruct((E, HALF), jnp.float32),
        ],
    )(edge_attr, wet, bev)

    agg0, agg1 = _sc_agg(src, dst, p, xw0, xw1, ef0, ef1)

    rep = jnp.kron(jnp.eye(H, dtype=jnp.float32),
                   jnp.ones((1, C), jnp.float32))
    out = pl.pallas_call(
        _combine_body,
        grid=(N // bn,),
        in_specs=[
            pl.BlockSpec((1, bn, HALF), lambda i: (0, i, 0)),
            pl.BlockSpec((1, bn, HALF), lambda i: (1, i, 0)),
            pl.BlockSpec((1, bn, HALF), lambda i: (0, i, 0)),
            pl.BlockSpec((1, bn, HALF), lambda i: (1, i, 0)),
            pl.BlockSpec((bn, H), lambda i: (i, 0)),
            pl.BlockSpec((H, HC), lambda i: (0, 0)),
            pl.BlockSpec((bn, HC), lambda i: (i, 0)),
        ],
        out_specs=pl.BlockSpec((bn, HC), lambda i: (i, 0)),
        out_shape=jax.ShapeDtypeStruct((N, HC), jnp.float32),
    )(agg0, agg0, agg1, agg1, ss, rep, res)
    return out


# speedup vs baseline: 1.1975x; 1.1975x over previous
import functools

import jax
import jax.numpy as jnp
from jax import lax
from jax.experimental import pallas as pl
from jax.experimental.pallas import tpu as pltpu
from jax.experimental.pallas import tpu_sc as plsc

N = 10000
E = 160000
D = 256
H = 16
C = 16
HC = 256
TD = 16
HALF = 128

NC = 2
NS = 16
NW = NC * NS
B = 128
NCHUNK = E // B
KMAX = -(-NCHUNK // NW)
B2 = 64
NCHUNK2 = E // B2
KMAX2 = 80
NP = NW * 314
ZROWS = NP // NS
BE_AB = 1280

_mesh = plsc.VectorSubcoreMesh(core_axis_name="c", subcore_axis_name="s",
                               num_cores=NC, num_subcores=NS)


def _node_body(x_ref, wlt_ref, wrt_ref, plm_ref, prm_ref,
               xw0_ref, xw1_ref, res_ref, al_ref, ar_ref):
    xb = x_ref[...]
    xw = jnp.dot(xb, wlt_ref[...], preferred_element_type=jnp.float32)
    res_ref[...] = jnp.dot(xb, wrt_ref[...], preferred_element_type=jnp.float32)
    xw0_ref[...] = xw[:, :HALF]
    xw1_ref[...] = xw[:, HALF:]
    al_ref[...] = jnp.dot(xw, plm_ref[...], preferred_element_type=jnp.float32)
    ar_ref[...] = jnp.dot(xw, prm_ref[...], preferred_element_type=jnp.float32)


def _ab_body(ea_ref, m_ref, c0_ref, ab_ref):
    ab_ref[...] = (jnp.dot(ea_ref[...], m_ref[...],
                           preferred_element_type=jnp.float32) + c0_ref[...])


def _ef_body(ea_ref, wet_ref, bev_ref, ef0_ref, ef1_ref):
    ef = jnp.dot(ea_ref[...], wet_ref[...], preferred_element_type=jnp.float32)
    ef = ef + bev_ref[...]
    ef0_ref[...] = ef[:, :HALF]
    ef1_ref[...] = ef[:, HALF:]


def _combine_body(a00_ref, a01_ref, a10_ref, a11_ref, ss_ref, rep_ref,
                  res_ref, out_ref):
    s0 = a00_ref[0] + a01_ref[0]
    s1 = a10_ref[0] + a11_ref[0]
    agg = jnp.concatenate([s0, s1], axis=1)
    den = jnp.dot(ss_ref[...], rep_ref[...],
                  preferred_element_type=jnp.float32) + 1e-16
    agg = agg / den
    out_ref[...] = jnp.where(agg > 0.0, agg, jnp.exp(agg) - 1.0) + res_ref[...]


def _sc_softmax_body(src_hbm, dst_hbm, al_hbm, ar_hbm, ab_hbm, ew_hbm,
                     aev_hbm, p_hbm, part_hbm,
                     isrc0, isrc1, idst0, idst1, gl0, gl1, gr0, gr1,
                     ab0, ab1, ew0, ew1, aev_v, p_v, ps_v, acc_sh,
                     sem0, sem1):
    isrc_v = [isrc0, isrc1]
    idst_v = [idst0, idst1]
    gl_v = [gl0, gl1]
    gr_v = [gr0, gr1]
    ab_v = [ab0, ab1]
    ew_v = [ew0, ew1]
    sem = [sem0, sem1]
    pltpu.sync_copy(aev_hbm, aev_v)
    av = aev_v[...]
    cid = lax.axis_index("c")
    sid = lax.axis_index("s")
    wid = sid * NC + cid
    zf = jnp.zeros((H,), jnp.float32)

    def zinit(j, _):
        gl_v[0][j] = zf
        return ()
    lax.fori_loop(0, B, zinit, ())
    for q in range(4):
        pltpu.sync_copy(gl_v[0], acc_sh.at[pl.ds(sid * ZROWS + q * B, B)])
    pltpu.sync_copy(gl_v[0].at[pl.ds(0, ZROWS - 4 * B)],
                    acc_sh.at[pl.ds(sid * ZROWS + 4 * B, ZROWS - 4 * B)])
    plsc.subcore_barrier()

    def issue(k, b):
        g = k * NW + wid

        @pl.when(g < NCHUNK)
        def _():
            base = g * B
            pltpu.sync_copy(src_hbm.at[pl.ds(base, B)], isrc_v[b])
            pltpu.sync_copy(dst_hbm.at[pl.ds(base, B)], idst_v[b])
            pltpu.async_copy(al_hbm.at[isrc_v[b]], gl_v[b], sem[b])
            pltpu.async_copy(ar_hbm.at[idst_v[b]], gr_v[b], sem[b])
            pltpu.async_copy(ab_hbm.at[pl.ds(base, B)], ab_v[b], sem[b])
            pltpu.async_copy(ew_hbm.at[pl.ds(base, B)], ew_v[b], sem[b])

    def consume(k, b):
        g = k * NW + wid

        @pl.when(g < NCHUNK)
        def _():
            base = g * B
            pltpu.make_async_copy(al_hbm.at[isrc_v[b]], gl_v[b], sem[b]).wait()
            pltpu.make_async_copy(ar_hbm.at[idst_v[b]], gr_v[b], sem[b]).wait()
            pltpu.make_async_copy(ab_hbm.at[pl.ds(base, B)], ab_v[b],
                                  sem[b]).wait()
            pltpu.make_async_copy(ew_hbm.at[pl.ds(base, B)], ew_v[b],
                                  sem[b]).wait()

            def edge16(j16, _):
                rw = ew_v[b][pl.ds(j16 * H, H)]
                for t in range(H):
                    j = j16 * H + t
                    s = gl_v[b][j] + gr_v[b][j] + ab_v[b][j] + rw[t] * av
                    a = jnp.where(s >= 0.0, s, 0.2 * s)
                    pe = jnp.exp(a)
                    p_v[pl.ds(j * H, H)] = pe
                    ps_v[j] = pe
                return ()
            lax.fori_loop(0, B // H, edge16, ())

            pltpu.sync_copy(p_v, p_hbm.at[pl.ds(base * H, B * H)])
            pltpu.sync_copy(ps_v, acc_sh.at[idst_v[b]], add=True)

    issue(0, 0)
    issue(1, 1)

    def pair(ii, _):
        for par in range(2):
            k = ii * 2 + par
            consume(k, par)
            issue(k + 2, par)
        return ()
    lax.fori_loop(0, KMAX // 2, pair, ())
    plsc.subcore_barrier()

    @pl.when(sid == 0)
    def _():
        pltpu.sync_copy(acc_sh, part_hbm.at[cid])


def _sc_agg_body(src_hbm, dst_hbm, p_hbm, xw0_hbm, xw1_hbm,
                 ef0_hbm, ef1_hbm, agg0_hbm, agg1_hbm,
                 isrc0, isrc1, idst0, idst1, xg0, xg1, ef0_v, ef1_v,
                 p0, p1, acc_sh, sem0, sem1):
    isrc_v = [isrc0, isrc1]
    idst_v = [idst0, idst1]
    xg_v = [xg0, xg1]
    ef_v = [ef0_v, ef1_v]
    p_v = [p0, p1]
    sem = [sem0, sem1]
    cid = lax.axis_index("c")
    sid = lax.axis_index("s")
    wid = sid * NC + cid
    zf = jnp.zeros((C,), jnp.float32)

    for half in range(2):
        xwh = xw0_hbm if half == 0 else xw1_hbm
        efh = ef0_hbm if half == 0 else ef1_hbm
        aggh = agg0_hbm if half == 0 else agg1_hbm

        def zinit(j, _):
            for c8 in range(8):
                xg0[j, pl.ds(c8 * 16, 16)] = zf
            return ()
        lax.fori_loop(0, B2, zinit, ())
        for q in range(9):
            pltpu.sync_copy(xg0, acc_sh.at[pl.ds(sid * ZROWS + q * B2, B2)])
        pltpu.sync_copy(xg0.at[pl.ds(0, ZROWS - 9 * B2)],
                        acc_sh.at[pl.ds(sid * ZROWS + 9 * B2, ZROWS - 9 * B2)])
        plsc.subcore_barrier()

        def issue(k, b):
            g = k * NW + wid

            @pl.when(g < NCHUNK2)
            def _():
                base = g * B2
                pltpu.sync_copy(src_hbm.at[pl.ds(base, B2)], isrc_v[b])
                pltpu.sync_copy(dst_hbm.at[pl.ds(base, B2)], idst_v[b])
                pltpu.async_copy(xwh.at[isrc_v[b]], xg_v[b], sem[b])
                pltpu.async_copy(efh.at[pl.ds(base, B2)], ef_v[b], sem[b])
                pltpu.async_copy(p_hbm.at[pl.ds(base * H, B2 * H)], p_v[b],
                                 sem[b])

        def consume(k, b):
            g = k * NW + wid

            @pl.when(g < NCHUNK2)
            def _():
                base = g * B2
                pltpu.make_async_copy(xwh.at[isrc_v[b]], xg_v[b],
                                      sem[b]).wait()
                pltpu.make_async_copy(efh.at[pl.ds(base, B2)], ef_v[b],
                                      sem[b]).wait()
                pltpu.make_async_copy(p_hbm.at[pl.ds(base * H, B2 * H)],
                                      p_v[b], sem[b]).wait()

                def edge(j, _):
                    co = p_v[b][pl.ds(j * H, H)]
                    for hh in range(8):
                        sc = co[half * 8 + hh]
                        sl = pl.ds(hh * 16, 16)
                        xg_v[b][j, sl] = (xg_v[b][j, sl] + ef_v[b][j, sl]) * sc
                    return ()
                lax.fori_loop(0, B2, edge, ())

                pltpu.sync_copy(xg_v[b], acc_sh.at[idst_v[b]], add=True)

        issue(0, 0)
        issue(1, 1)

        def pair(ii, _):
            for par in range(2):
                k = ii * 2 + par
                consume(k, par)
                issue(k + 2, par)
            return ()
        lax.fori_loop(0, KMAX2 // 2, pair, ())
        plsc.subcore_barrier()

        @pl.when(sid == 0)
        def _():
            pltpu.sync_copy(acc_sh, aggh.at[cid])
        plsc.subcore_barrier()


_sc_softmax = pl.kernel(
    _sc_softmax_body,
    out_type=(jax.ShapeDtypeStruct((E * H,), jnp.float32),
              jax.ShapeDtypeStruct((NC, NP, H), jnp.float32)),
    mesh=_mesh,
    scratch_types=[
        pltpu.VMEM((B,), jnp.int32),
        pltpu.VMEM((B,), jnp.int32),
        pltpu.VMEM((B,), jnp.int32),
        pltpu.VMEM((B,), jnp.int32),
        pltpu.VMEM((B, H), jnp.float32),
        pltpu.VMEM((B, H), jnp.float32),
        pltpu.VMEM((B, H), jnp.float32),
        pltpu.VMEM((B, H), jnp.float32),
        pltpu.VMEM((B, H), jnp.float32),
        pltpu.VMEM((B, H), jnp.float32),
        pltpu.VMEM((B,), jnp.float32),
        pltpu.VMEM((B,), jnp.float32),
        pltpu.VMEM((H,), jnp.float32),
        pltpu.VMEM((B * H,), jnp.float32),
        pltpu.VMEM((B, H), jnp.float32),
        pltpu.VMEM_SHARED((NP, H), jnp.float32),
        pltpu.SemaphoreType.DMA,
        pltpu.SemaphoreType.DMA,
    ],
    compiler_params=pltpu.CompilerParams(use_tc_tiling_on_sc=False),
)

_sc_agg = pl.kernel(
    _sc_agg_body,
    out_type=(jax.ShapeDtypeStruct((NC, NP, HALF), jnp.float32),
              jax.ShapeDtypeStruct((NC, NP, HALF), jnp.float32)),
    mesh=_mesh,
    scratch_types=[
        pltpu.VMEM((B2,), jnp.int32),
        pltpu.VMEM((B2,), jnp.int32),
        pltpu.VMEM((B2,), jnp.int32),
        pltpu.VMEM((B2,), jnp.int32),
        pltpu.VMEM((B2, HALF), jnp.float32),
        pltpu.VMEM((B2, HALF), jnp.float32),
        pltpu.VMEM((B2, HALF), jnp.float32),
        pltpu.VMEM((B2, HALF), jnp.float32),
        pltpu.VMEM((B2 * H,), jnp.float32),
        pltpu.VMEM((B2 * H,), jnp.float32),
        pltpu.VMEM_SHARED((NP, HALF), jnp.float32),
        pltpu.SemaphoreType.DMA,
        pltpu.SemaphoreType.DMA,
    ],
)


@jax.jit
def kernel(x, edge_index, edge_weight, edge_attr, W_lin, att_l, att_r, att_e,
           W_e, b_e, W_res):
    src = edge_index[0]
    dst = edge_index[1]

    eye = jnp.eye(H, dtype=jnp.float32)
    plm = (att_l[0][:, :, None] * eye[:, None, :]).reshape(HC, H)
    prm = (att_r[0][:, :, None] * eye[:, None, :]).reshape(HC, H)
    aev = att_e.reshape(1, H)
    bev = b_e.reshape(1, HC)
    ew2 = edge_weight.reshape(E)

    bn = 400
    xw0, xw1, res, al, ar = pl.pallas_call(
        _node_body,
        grid=(N // bn,),
        in_specs=[
            pl.BlockSpec((bn, D), lambda i: (i, 0)),
            pl.BlockSpec((D, HC), lambda i: (0, 0)),
            pl.BlockSpec((D, HC), lambda i: (0, 0)),
            pl.BlockSpec((HC, H), lambda i: (0, 0)),
            pl.BlockSpec((HC, H), lambda i: (0, 0)),
        ],
        out_specs=[
            pl.BlockSpec((bn, HALF), lambda i: (i, 0)),
            pl.BlockSpec((bn, HALF), lambda i: (i, 0)),
            pl.BlockSpec((bn, HC), lambda i: (i, 0)),
            pl.BlockSpec((bn, H), lambda i: (i, 0)),
            pl.BlockSpec((bn, H), lambda i: (i, 0)),
        ],
        out_shape=[
            jax.ShapeDtypeStruct((N, HALF), jnp.float32),
            jax.ShapeDtypeStruct((N, HALF), jnp.float32),
            jax.ShapeDtypeStruct((N, HC), jnp.float32),
            jax.ShapeDtypeStruct((N, H), jnp.float32),
            jax.ShapeDtypeStruct((N, H), jnp.float32),
        ],
    )(x, W_lin.T, W_res.T, plm, prm)

    wet = W_e.T
    m_comb = jnp.dot(wet, plm)
    c0 = jnp.dot(bev, plm)
    be = 2000
    ab = pl.pallas_call(
        _ab_body,
        grid=(E // be,),
        in_specs=[
            pl.BlockSpec((be, TD), lambda i: (i, 0)),
            pl.BlockSpec((TD, H), lambda i: (0, 0)),
            pl.BlockSpec((1, H), lambda i: (0, 0)),
        ],
        out_specs=pl.BlockSpec((be, H), lambda i: (i, 0)),
        out_shape=jax.ShapeDtypeStruct((E, H), jnp.float32),
    )(edge_attr, m_comb, c0)

    p, parts = _sc_softmax(src, dst, al, ar, ab, ew2, att_e.reshape(H))
    ss = parts[0] + parts[1]

    ef0, ef1 = pl.pallas_call(
        _ef_body,
        grid=(E // be,),
        in_specs=[
            pl.BlockSpec((be, TD), lambda i: (i, 0)),
            pl.BlockSpec((TD, HC), lambda i: (0, 0)),
            pl.BlockSpec((1, HC), lambda i: (0, 0)),
        ],
        out_specs=[
            pl.BlockSpec((be, HALF), lambda i: (i, 0)),
            pl.BlockSpec((be, HALF), lambda i: (i, 0)),
        ],
        out_shape=[
            jax.ShapeDtypeStruct((E, HALF), jnp.float32),
            jax.ShapeDtypeStruct((E, HALF), jnp.float32),
        ],
    )(edge_attr, wet, bev)

    agg0, agg1 = _sc_agg(src, dst, p, xw0, xw1, ef0, ef1)

    rep = jnp.kron(jnp.eye(H, dtype=jnp.float32),
                   jnp.ones((1, C), jnp.float32))
    out = pl.pallas_call(
        _combine_body,
        grid=(N // bn,),
        in_specs=[
            pl.BlockSpec((1, bn, HALF), lambda i: (0, i, 0)),
            pl.BlockSpec((1, bn, HALF), lambda i: (1, i, 0)),
            pl.BlockSpec((1, bn, HALF), lambda i: (0, i, 0)),
            pl.BlockSpec((1, bn, HALF), lambda i: (1, i, 0)),
            pl.BlockSpec((bn, H), lambda i: (i, 0)),
            pl.BlockSpec((H, HC), lambda i: (0, 0)),
            pl.BlockSpec((bn, HC), lambda i: (i, 0)),
        ],
        out_specs=pl.BlockSpec((bn, HC), lambda i: (i, 0)),
        out_shape=jax.ShapeDtypeStruct((N, HC), jnp.float32),
    )(agg0, agg0, agg1, agg1, ss, rep, res)
    return out


# speedup vs baseline: 1.2420x; 1.0372x over previous
import functools

import jax
import jax.numpy as jnp
from jax import lax
from jax.experimental import pallas as pl
from jax.experimental.pallas import tpu as pltpu
from jax.experimental.pallas import tpu_sc as plsc

N = 10000
E = 160000
D = 256
H = 16
C = 16
HC = 256
TD = 16
HALF = 128

NC = 2
NS = 16
NW = NC * NS
B = 128
NCHUNK = E // B
KMAX = -(-NCHUNK // NW)
B2 = 64
NCHUNK2 = E // B2
KMAX2 = 80
NP = NW * 314
ZROWS = NP // NS
BE_AB = 1280

_mesh = plsc.VectorSubcoreMesh(core_axis_name="c", subcore_axis_name="s",
                               num_cores=NC, num_subcores=NS)


def _node_body(x_ref, wlt_ref, wrt_ref, plm_ref, prm_ref,
               xw0_ref, xw1_ref, res_ref, al_ref, ar_ref):
    xb = x_ref[...]
    xw = jnp.dot(xb, wlt_ref[...], preferred_element_type=jnp.float32)
    res_ref[...] = jnp.dot(xb, wrt_ref[...], preferred_element_type=jnp.float32)
    xw0_ref[...] = xw[:, :HALF]
    xw1_ref[...] = xw[:, HALF:]
    al_ref[...] = jnp.dot(xw, plm_ref[...], preferred_element_type=jnp.float32)
    ar_ref[...] = jnp.dot(xw, prm_ref[...], preferred_element_type=jnp.float32)


def _ab_body(eag_ref, m2_ref, c0t_ref, ab_ref):
    ab_ref[...] = (jnp.dot(eag_ref[...], m2_ref[...],
                           preferred_element_type=jnp.float32) + c0t_ref[...])


def _ef_body(ea_ref, wet_ref, bev_ref, ef0_ref, ef1_ref):
    ef = jnp.dot(ea_ref[...], wet_ref[...], preferred_element_type=jnp.float32)
    ef = ef + bev_ref[...]
    ef0_ref[...] = ef[:, :HALF]
    ef1_ref[...] = ef[:, HALF:]


def _combine_body(a00_ref, a01_ref, a10_ref, a11_ref, ss_ref, rep_ref,
                  res_ref, out_ref):
    s0 = a00_ref[0] + a01_ref[0]
    s1 = a10_ref[0] + a11_ref[0]
    agg = jnp.concatenate([s0, s1], axis=1)
    den = jnp.dot(ss_ref[...], rep_ref[...],
                  preferred_element_type=jnp.float32) + 1e-16
    agg = agg / den
    out_ref[...] = jnp.where(agg > 0.0, agg, jnp.exp(agg) - 1.0) + res_ref[...]


def _sc_softmax_body(src_hbm, dst_hbm, al_hbm, ar_hbm, ab_hbm, ew_hbm,
                     aev_hbm, p_hbm, part_hbm,
                     isrc0, isrc1, idst0, idst1, gl0, gl1, gr0, gr1,
                     ab0, ab1, ew0, ew1, aev_v, p_v, ps_v, acc_sh,
                     sem0, sem1):
    isrc_v = [isrc0, isrc1]
    idst_v = [idst0, idst1]
    gl_v = [gl0, gl1]
    gr_v = [gr0, gr1]
    ab_v = [ab0, ab1]
    ew_v = [ew0, ew1]
    sem = [sem0, sem1]
    pltpu.sync_copy(aev_hbm, aev_v)
    av = aev_v[...]
    cid = lax.axis_index("c")
    sid = lax.axis_index("s")
    wid = sid * NC + cid
    zf = jnp.zeros((H,), jnp.float32)

    def zinit(j, _):
        gl_v[0][j] = zf
        return ()
    lax.fori_loop(0, B, zinit, ())
    for q in range(4):
        pltpu.sync_copy(gl_v[0], acc_sh.at[pl.ds(sid * ZROWS + q * B, B)])
    pltpu.sync_copy(gl_v[0].at[pl.ds(0, ZROWS - 4 * B)],
                    acc_sh.at[pl.ds(sid * ZROWS + 4 * B, ZROWS - 4 * B)])
    plsc.subcore_barrier()

    def issue(k, b):
        g = k * NW + wid

        @pl.when(g < NCHUNK)
        def _():
            base = g * B
            pltpu.sync_copy(src_hbm.at[pl.ds(base, B)], isrc_v[b])
            pltpu.sync_copy(dst_hbm.at[pl.ds(base, B)], idst_v[b])
            pltpu.async_copy(al_hbm.at[isrc_v[b]], gl_v[b], sem[b])
            pltpu.async_copy(ar_hbm.at[idst_v[b]], gr_v[b], sem[b])
            pltpu.async_copy(ab_hbm.at[pl.ds(base * H, B * H)], ab_v[b],
                             sem[b])
            pltpu.async_copy(ew_hbm.at[pl.ds(base, B)], ew_v[b], sem[b])

    def consume(k, b):
        g = k * NW + wid

        @pl.when(g < NCHUNK)
        def _():
            base = g * B
            pltpu.make_async_copy(al_hbm.at[isrc_v[b]], gl_v[b], sem[b]).wait()
            pltpu.make_async_copy(ar_hbm.at[idst_v[b]], gr_v[b], sem[b]).wait()
            pltpu.make_async_copy(ab_hbm.at[pl.ds(base * H, B * H)], ab_v[b],
                                  sem[b]).wait()
            pltpu.make_async_copy(ew_hbm.at[pl.ds(base, B)], ew_v[b],
                                  sem[b]).wait()

            def edge16(j16, _):
                rw = ew_v[b][pl.ds(j16 * H, H)]
                for t in range(H):
                    j = j16 * H + t
                    s = (gl_v[b][j] + gr_v[b][j] + ab_v[b][pl.ds(j * H, H)]
                         + rw[t] * av)
                    a = jnp.where(s >= 0.0, s, 0.2 * s)
                    pe = jnp.exp(a)
                    p_v[pl.ds(j * H, H)] = pe
                    ps_v[j] = pe
                return ()
            lax.fori_loop(0, B // H, edge16, ())

            pltpu.sync_copy(p_v, p_hbm.at[pl.ds(base * H, B * H)])
            pltpu.sync_copy(ps_v, acc_sh.at[idst_v[b]], add=True)

    issue(0, 0)
    issue(1, 1)

    def pair(ii, _):
        for par in range(2):
            k = ii * 2 + par
            consume(k, par)
            issue(k + 2, par)
        return ()
    lax.fori_loop(0, KMAX // 2, pair, ())
    plsc.subcore_barrier()

    @pl.when(sid == 0)
    def _():
        pltpu.sync_copy(acc_sh, part_hbm.at[cid])


def _sc_agg_body(src_hbm, dst_hbm, p_hbm, xw0_hbm, xw1_hbm,
                 ef0_hbm, ef1_hbm, agg0_hbm, agg1_hbm,
                 isrc0, isrc1, idst0, idst1, xg0, xg1, ef0_v, ef1_v,
                 p0, p1, acc_sh, sem0, sem1):
    isrc_v = [isrc0, isrc1]
    idst_v = [idst0, idst1]
    xg_v = [xg0, xg1]
    ef_v = [ef0_v, ef1_v]
    p_v = [p0, p1]
    sem = [sem0, sem1]
    cid = lax.axis_index("c")
    sid = lax.axis_index("s")
    wid = sid * NC + cid
    zf = jnp.zeros((C,), jnp.float32)

    for half in range(2):
        xwh = xw0_hbm if half == 0 else xw1_hbm
        efh = ef0_hbm if half == 0 else ef1_hbm
        aggh = agg0_hbm if half == 0 else agg1_hbm

        def zinit(j, _):
            for c8 in range(8):
                xg0[j, pl.ds(c8 * 16, 16)] = zf
            return ()
        lax.fori_loop(0, B2, zinit, ())
        for q in range(9):
            pltpu.sync_copy(xg0, acc_sh.at[pl.ds(sid * ZROWS + q * B2, B2)])
        pltpu.sync_copy(xg0.at[pl.ds(0, ZROWS - 9 * B2)],
                        acc_sh.at[pl.ds(sid * ZROWS + 9 * B2, ZROWS - 9 * B2)])
        plsc.subcore_barrier()

        def issue(k, b):
            g = k * NW + wid

            @pl.when(g < NCHUNK2)
            def _():
                base = g * B2
                pltpu.sync_copy(src_hbm.at[pl.ds(base, B2)], isrc_v[b])
                pltpu.sync_copy(dst_hbm.at[pl.ds(base, B2)], idst_v[b])
                pltpu.async_copy(xwh.at[isrc_v[b]], xg_v[b], sem[b])
                pltpu.async_copy(efh.at[pl.ds(base, B2)], ef_v[b], sem[b])
                pltpu.async_copy(p_hbm.at[pl.ds(base * H, B2 * H)], p_v[b],
                                 sem[b])

        def consume(k, b):
            g = k * NW + wid

            @pl.when(g < NCHUNK2)
            def _():
                base = g * B2
                pltpu.make_async_copy(xwh.at[isrc_v[b]], xg_v[b],
                                      sem[b]).wait()
                pltpu.make_async_copy(efh.at[pl.ds(base, B2)], ef_v[b],
                                      sem[b]).wait()
                pltpu.make_async_copy(p_hbm.at[pl.ds(base * H, B2 * H)],
                                      p_v[b], sem[b]).wait()

                def edge(j, _):
                    co = p_v[b][pl.ds(j * H, H)]
                    for hh in range(8):
                        sc = co[half * 8 + hh]
                        sl = pl.ds(hh * 16, 16)
                        xg_v[b][j, sl] = (xg_v[b][j, sl] + ef_v[b][j, sl]) * sc
                    return ()
                lax.fori_loop(0, B2, edge, ())

                pltpu.sync_copy(xg_v[b], acc_sh.at[idst_v[b]], add=True)

        issue(0, 0)
        issue(1, 1)

        def pair(ii, _):
            for par in range(2):
                k = ii * 2 + par
                consume(k, par)
                issue(k + 2, par)
            return ()
        lax.fori_loop(0, KMAX2 // 2, pair, ())
        plsc.subcore_barrier()

        @pl.when(sid == 0)
        def _():
            pltpu.sync_copy(acc_sh, aggh.at[cid])
        plsc.subcore_barrier()


_sc_softmax = pl.kernel(
    _sc_softmax_body,
    out_type=(jax.ShapeDtypeStruct((E * H,), jnp.float32),
              jax.ShapeDtypeStruct((NC, NP, H), jnp.float32)),
    mesh=_mesh,
    scratch_types=[
        pltpu.VMEM((B,), jnp.int32),
        pltpu.VMEM((B,), jnp.int32),
        pltpu.VMEM((B,), jnp.int32),
        pltpu.VMEM((B,), jnp.int32),
        pltpu.VMEM((B, H), jnp.float32),
        pltpu.VMEM((B, H), jnp.float32),
        pltpu.VMEM((B, H), jnp.float32),
        pltpu.VMEM((B, H), jnp.float32),
        pltpu.VMEM((B * H,), jnp.float32),
        pltpu.VMEM((B * H,), jnp.float32),
        pltpu.VMEM((B,), jnp.float32),
        pltpu.VMEM((B,), jnp.float32),
        pltpu.VMEM((H,), jnp.float32),
        pltpu.VMEM((B * H,), jnp.float32),
        pltpu.VMEM((B, H), jnp.float32),
        pltpu.VMEM_SHARED((NP, H), jnp.float32),
        pltpu.SemaphoreType.DMA,
        pltpu.SemaphoreType.DMA,
    ],
    compiler_params=pltpu.CompilerParams(use_tc_tiling_on_sc=False),
)

_sc_agg = pl.kernel(
    _sc_agg_body,
    out_type=(jax.ShapeDtypeStruct((NC, NP, HALF), jnp.float32),
              jax.ShapeDtypeStruct((NC, NP, HALF), jnp.float32)),
    mesh=_mesh,
    scratch_types=[
        pltpu.VMEM((B2,), jnp.int32),
        pltpu.VMEM((B2,), jnp.int32),
        pltpu.VMEM((B2,), jnp.int32),
        pltpu.VMEM((B2,), jnp.int32),
        pltpu.VMEM((B2, HALF), jnp.float32),
        pltpu.VMEM((B2, HALF), jnp.float32),
        pltpu.VMEM((B2, HALF), jnp.float32),
        pltpu.VMEM((B2, HALF), jnp.float32),
        pltpu.VMEM((B2 * H,), jnp.float32),
        pltpu.VMEM((B2 * H,), jnp.float32),
        pltpu.VMEM_SHARED((NP, HALF), jnp.float32),
        pltpu.SemaphoreType.DMA,
        pltpu.SemaphoreType.DMA,
    ],
)


@jax.jit
def kernel(x, edge_index, edge_weight, edge_attr, W_lin, att_l, att_r, att_e,
           W_e, b_e, W_res):
    src = edge_index[0]
    dst = edge_index[1]

    eye = jnp.eye(H, dtype=jnp.float32)
    plm = (att_l[0][:, :, None] * eye[:, None, :]).reshape(HC, H)
    prm = (att_r[0][:, :, None] * eye[:, None, :]).reshape(HC, H)
    aev = att_e.reshape(1, H)
    bev = b_e.reshape(1, HC)
    ew2 = edge_weight.reshape(E)

    bn = 400
    xw0, xw1, res, al, ar = pl.pallas_call(
        _node_body,
        grid=(N // bn,),
        in_specs=[
            pl.BlockSpec((bn, D), lambda i: (i, 0)),
            pl.BlockSpec((D, HC), lambda i: (0, 0)),
            pl.BlockSpec((D, HC), lambda i: (0, 0)),
            pl.BlockSpec((HC, H), lambda i: (0, 0)),
            pl.BlockSpec((HC, H), lambda i: (0, 0)),
        ],
        out_specs=[
            pl.BlockSpec((bn, HALF), lambda i: (i, 0)),
            pl.BlockSpec((bn, HALF), lambda i: (i, 0)),
            pl.BlockSpec((bn, HC), lambda i: (i, 0)),
            pl.BlockSpec((bn, H), lambda i: (i, 0)),
            pl.BlockSpec((bn, H), lambda i: (i, 0)),
        ],
        out_shape=[
            jax.ShapeDtypeStruct((N, HALF), jnp.float32),
            jax.ShapeDtypeStruct((N, HALF), jnp.float32),
            jax.ShapeDtypeStruct((N, HC), jnp.float32),
            jax.ShapeDtypeStruct((N, H), jnp.float32),
            jax.ShapeDtypeStruct((N, H), jnp.float32),
        ],
    )(x, W_lin.T, W_res.T, plm, prm)

    wet = W_e.T
    m_comb = jnp.dot(wet, plm)
    c0 = jnp.dot(bev, plm)
    be = 2000
    eag = edge_attr.reshape(E // 8, 8 * TD)
    m2 = jnp.kron(jnp.eye(8, dtype=jnp.float32), m_comb)
    c0t = jnp.tile(c0, (1, 8))
    gr_ab = 200
    ab2 = pl.pallas_call(
        _ab_body,
        grid=(E // 8 // gr_ab,),
        in_specs=[
            pl.BlockSpec((gr_ab, 8 * TD), lambda i: (i, 0)),
            pl.BlockSpec((8 * TD, 8 * H), lambda i: (0, 0)),
            pl.BlockSpec((1, 8 * H), lambda i: (0, 0)),
        ],
        out_specs=pl.BlockSpec((gr_ab, 8 * H), lambda i: (i, 0)),
        out_shape=jax.ShapeDtypeStruct((E // 8, 8 * H), jnp.float32),
    )(eag, m2, c0t)
    ab = ab2.reshape(E * H)

    p, parts = _sc_softmax(src, dst, al, ar, ab, ew2, att_e.reshape(H))
    ss = parts[0] + parts[1]

    ef0, ef1 = pl.pallas_call(
        _ef_body,
        grid=(E // be,),
        in_specs=[
            pl.BlockSpec((be, TD), lambda i: (i, 0)),
            pl.BlockSpec((TD, HC), lambda i: (0, 0)),
            pl.BlockSpec((1, HC), lambda i: (0, 0)),
        ],
        out_specs=[
            pl.BlockSpec((be, HALF), lambda i: (i, 0)),
            pl.BlockSpec((be, HALF), lambda i: (i, 0)),
        ],
        out_shape=[
            jax.ShapeDtypeStruct((E, HALF), jnp.float32),
            jax.ShapeDtypeStruct((E, HALF), jnp.float32),
        ],
    )(edge_attr, wet, bev)

    agg0, agg1 = _sc_agg(src, dst, p, xw0, xw1, ef0, ef1)

    rep = jnp.kron(jnp.eye(H, dtype=jnp.float32),
                   jnp.ones((1, C), jnp.float32))
    out = pl.pallas_call(
        _combine_body,
        grid=(N // bn,),
        in_specs=[
            pl.BlockSpec((1, bn, HALF), lambda i: (0, i, 0)),
            pl.BlockSpec((1, bn, HALF), lambda i: (1, i, 0)),
            pl.BlockSpec((1, bn, HALF), lambda i: (0, i, 0)),
            pl.BlockSpec((1, bn, HALF), lambda i: (1, i, 0)),
            pl.BlockSpec((bn, H), lambda i: (i, 0)),
            pl.BlockSpec((H, HC), lambda i: (0, 0)),
            pl.BlockSpec((bn, HC), lambda i: (i, 0)),
        ],
        out_specs=pl.BlockSpec((bn, HC), lambda i: (i, 0)),
        out_shape=jax.ShapeDtypeStruct((N, HC), jnp.float32),
    )(agg0, agg0, agg1, agg1, ss, rep, res)
    return out
